# Initial kernel scaffold; baseline (speedup 1.0000x reference)
import jax
import jax.numpy as jnp
from jax.experimental import pallas as pl


def kernel(y_large, y_small, wids_large, wids_small, lora_B_large, lora_B_small):
    raise NotImplementedError("write your pallas kernel here")


# speedup vs baseline: 2.1341x; 2.1341x over previous
import functools

import jax
import jax.numpy as jnp
from jax.experimental import pallas as pl


NT = 512


def _matmul_body(yl_ref, ys_ref, wl_ref, ws_ref, bl_ref, bs_ref, out_ref):
    yl = yl_ref[...].astype(jnp.bfloat16)
    ys = ys_ref[...].astype(jnp.bfloat16)
    wl = wl_ref[...]
    ws = ws_ref[...]

    ysc_l = jnp.concatenate([yl] * 16, axis=1)
    aid_l = jax.lax.broadcasted_iota(jnp.int32, (128, 1024), 1) >> 6
    ysc_l = jnp.where(aid_l == wl, ysc_l, jnp.bfloat16(0))

    ysc_s = jnp.concatenate([ys] * 64, axis=1)
    aid_s = jax.lax.broadcasted_iota(jnp.int32, (128, 1024), 1) >> 4
    ysc_s = jnp.where(aid_s == ws, ysc_s, jnp.bfloat16(0))

    zl = jax.lax.dot_general(ysc_l, bl_ref[...], (((1,), (0,)), ((), ())),
                             preferred_element_type=jnp.float32)
    zs = jax.lax.dot_general(ysc_s, bs_ref[...], (((1,), (0,)), ((), ())),
                             preferred_element_type=jnp.float32)
    out_ref[0:128, :] = zl * 2.0
    out_ref[128:256, :] = zs * 2.0


@jax.jit
def kernel(y_large, y_small, wids_large, wids_small, lora_B_large, lora_B_small):
    yl = y_large.reshape(128, 64).astype(jnp.float32)
    ys = y_small.reshape(128, 16).astype(jnp.float32)
    wl = wids_large.reshape(128, 1)
    ws = wids_small.reshape(128, 1)
    bl = lora_B_large.reshape(16 * 64, 4096).astype(jnp.bfloat16)
    bs = lora_B_small.reshape(64 * 16, 4096).astype(jnp.bfloat16)

    grid = 4096 // NT
    out = pl.pallas_call(
        _matmul_body,
        grid=(grid,),
        in_specs=[
            pl.BlockSpec((128, 64), lambda n: (0, 0)),
            pl.BlockSpec((128, 16), lambda n: (0, 0)),
            pl.BlockSpec((128, 1), lambda n: (0, 0)),
            pl.BlockSpec((128, 1), lambda n: (0, 0)),
            pl.BlockSpec((1024, NT), lambda n: (0, n)),
            pl.BlockSpec((1024, NT), lambda n: (0, n)),
        ],
        out_specs=pl.BlockSpec((256, NT), lambda n: (0, n)),
        out_shape=jax.ShapeDtypeStruct((256, 4096), jnp.float32),
    )(yl, ys, wl, ws, bl, bs)
    return out.astype(jnp.float16).reshape(256, 1, 4096)
